# baseline (speedup 1.0000x reference)
import functools

import jax
import jax.numpy as jnp
from jax import lax
from jax.experimental import pallas as pl
from jax.experimental.pallas import tpu as pltpu
from jax.experimental.pallas import tpu_sc as plsc

_K = 65536
_D = 128
_B = 16384
_NC = 2
_NS = 16
_NW = _NC * _NS
_RW = _K // _NW
_CH = 512
_NCH = _RW // _CH


@functools.partial(
    pl.kernel,
    out_type=jax.ShapeDtypeStruct((_K, _D), jnp.float32),
    mesh=plsc.VectorSubcoreMesh(core_axis_name="c", subcore_axis_name="s"),
    compiler_params=pltpu.CompilerParams(needs_layout_passes=False),
    scratch_types=[
        pltpu.VMEM((_B,), jnp.int32),
        pltpu.VMEM((_RW,), jnp.int32),
        pltpu.VMEM((_CH, _D), jnp.float32),
        pltpu.SemaphoreType.DMA,
    ],
)
def _sc_update(mem_hbm, x_hbm, y_hbm, out_hbm, ys, wtab, slab, sem):
    wid = lax.axis_index("s") * _NC + lax.axis_index("c")
    lo = wid * _RW
    hi = lo + _RW
    iota = lax.iota(jnp.int32, 16)

    pltpu.async_copy(y_hbm, ys, sem).wait()

    def initw(i, carry):
        wtab[pl.ds(i * 16, 16)] = jnp.full((16,), -1, jnp.int32)
        return carry

    lax.fori_loop(0, _RW // 16, initw, 0)

    def mark(i, carry):
        kv = ys[pl.ds(i * 16, 16)]
        mk = (kv >= lo) & (kv < hi)
        _, lastm = plsc.scan_count(kv, mask=mk)
        plsc.store_scatter(wtab, [kv - lo], i * 16 + iota, mask=mk & lastm)
        return carry

    lax.fori_loop(0, _B // 16, mark, 0)

    for c in range(_NCH):
        row0 = pl.multiple_of(lo + c * _CH, _CH)
        pltpu.async_copy(mem_hbm.at[pl.ds(row0, _CH)], slab, sem).wait()
        pltpu.async_copy(slab, out_hbm.at[pl.ds(row0, _CH)], sem).wait()


def kernel(memory, x, y):
    return _sc_update(memory, x, y)
